# baseline (speedup 1.0000x reference)
import jax
import jax.numpy as jnp
from jax import lax
from jax.experimental import pallas as pl
from jax.experimental.pallas import tpu as pltpu
from jax.experimental.pallas import tpu_sc as plsc

N = 10000
E = 320000
D = 128
D_OUT = 64
G = 64

NP = 10240
NC = 2
NS = 16
NW = NC * NS
CHUNK = 128
K = 80
EPT = K * CHUNK
E_PAD = NW * EPT
ROWS_PER_TILE = NP // NS
DEGW = 128


def _mk_mesh():
    return plsc.VectorSubcoreMesh(core_axis_name="c", subcore_axis_name="s")


def _deg_body(dstw_hbm, ones_hbm, zeros_hbm, out_hbm, dst_v, ones_v, acc_s, sem):
    c = lax.axis_index("c")
    s = lax.axis_index("s")
    wid = s * NC + c
    pltpu.sync_copy(dstw_hbm.at[wid], dst_v)
    pltpu.sync_copy(ones_hbm, ones_v)
    pltpu.sync_copy(zeros_hbm, acc_s.at[pl.ds(s * ROWS_PER_TILE, ROWS_PER_TILE)])
    plsc.subcore_barrier()

    def step(j, carry):
        pltpu.sync_copy(ones_v, acc_s.at[dst_v.at[j]], add=True)
        return carry

    lax.fori_loop(0, K, step, 0)
    plsc.subcore_barrier()
    sl = pl.ds(s * ROWS_PER_TILE, ROWS_PER_TILE)
    pltpu.sync_copy(acc_s.at[sl], out_hbm.at[c, sl])


def _deg_partials(dstw, ones_w, zeros_w):
    return pl.kernel(
        _deg_body,
        out_type=jax.ShapeDtypeStruct((NC, NP, DEGW), jnp.float32),
        mesh=_mk_mesh(),
        scratch_types=[
            pltpu.VMEM((K, CHUNK), jnp.int32),
            pltpu.VMEM((CHUNK, DEGW), jnp.float32),
            pltpu.VMEM_SHARED((NP, DEGW), jnp.float32),
            pltpu.SemaphoreType.DMA,
        ],
    )(dstw, ones_w, zeros_w)


def _agg_body(hp_hbm, srcw_hbm, dstw_hbm, zeros_hbm, out_hbm,
              src_v, dst_v, rows_v, acc_s, sem):
    c = lax.axis_index("c")
    s = lax.axis_index("s")
    wid = s * NC + c
    pltpu.sync_copy(srcw_hbm.at[wid], src_v)
    pltpu.sync_copy(dstw_hbm.at[wid], dst_v)
    pltpu.sync_copy(zeros_hbm, acc_s.at[pl.ds(s * ROWS_PER_TILE, ROWS_PER_TILE)])
    plsc.subcore_barrier()

    def step(j, carry):
        pltpu.async_copy(hp_hbm.at[src_v.at[j]], rows_v, sem).wait()
        pltpu.sync_copy(rows_v, acc_s.at[dst_v.at[j]], add=True)
        return carry

    lax.fori_loop(0, K, step, 0)
    plsc.subcore_barrier()
    sl = pl.ds(s * ROWS_PER_TILE, ROWS_PER_TILE)
    pltpu.sync_copy(acc_s.at[sl], out_hbm.at[c, sl])


def _agg_partials(hp, srcw, dstw, zeros_rows):
    return pl.kernel(
        _agg_body,
        out_type=jax.ShapeDtypeStruct((NC, NP, D), jnp.float32),
        mesh=_mk_mesh(),
        scratch_types=[
            pltpu.VMEM((K, CHUNK), jnp.int32),
            pltpu.VMEM((K, CHUNK), jnp.int32),
            pltpu.VMEM((CHUNK, D), jnp.float32),
            pltpu.VMEM_SHARED((NP, D), jnp.float32),
            pltpu.SemaphoreType.DMA,
        ],
    )(hp, srcw, dstw, zeros_rows)


_BM = 1024
_GRID = NP // _BM


def _mm_body(a_ref, w_ref, o_ref):
    o_ref[...] = jnp.dot(a_ref[...], w_ref[...], preferred_element_type=jnp.float32)


def _matmul(a, w):
    dout = w.shape[1]
    return pl.pallas_call(
        _mm_body,
        grid=(_GRID,),
        in_specs=[
            pl.BlockSpec((_BM, D), lambda i: (i, 0)),
            pl.BlockSpec((D, dout), lambda i: (0, 0)),
        ],
        out_specs=pl.BlockSpec((_BM, dout), lambda i: (i, 0)),
        out_shape=jax.ShapeDtypeStruct((NP, dout), jnp.float32),
    )(a, w)


def _dinv_body(degp_ref, h1m_ref, dinvb_ref, hp1_ref):
    i = pl.program_id(0)
    deg = 1.0 + degp_ref[0, :, 0:1] + degp_ref[1, :, 0:1]
    dinv = lax.rsqrt(deg)
    rid = lax.broadcasted_iota(jnp.int32, (_BM, 1), 0) + i * _BM
    dinv = jnp.where(rid < N, dinv, 0.0)
    dinvb = jnp.broadcast_to(dinv, (_BM, D))
    dinvb_ref[...] = dinvb
    hp1_ref[...] = h1m_ref[...] * dinvb


def _dinv_and_scale(degp, h1m):
    return pl.pallas_call(
        _dinv_body,
        grid=(_GRID,),
        in_specs=[
            pl.BlockSpec((NC, _BM, DEGW), lambda i: (0, i, 0)),
            pl.BlockSpec((_BM, D), lambda i: (i, 0)),
        ],
        out_specs=[
            pl.BlockSpec((_BM, D), lambda i: (i, 0)),
            pl.BlockSpec((_BM, D), lambda i: (i, 0)),
        ],
        out_shape=[
            jax.ShapeDtypeStruct((NP, D), jnp.float32),
            jax.ShapeDtypeStruct((NP, D), jnp.float32),
        ],
    )(degp, h1m)


def _layer_body(aggp_ref, hp_ref, dinvb_ref, b_ref, w_ref, o_ref):
    dinvb = dinvb_ref[...]
    h = (aggp_ref[0] + aggp_ref[1] + hp_ref[...]) * dinvb + b_ref[...]
    h = jnp.maximum(h, 0.0)
    o_ref[...] = jnp.dot(h, w_ref[...], preferred_element_type=jnp.float32) * dinvb


def _layer_combine(aggp, hp, dinvb, b, w):
    return pl.pallas_call(
        _layer_body,
        grid=(_GRID,),
        in_specs=[
            pl.BlockSpec((NC, _BM, D), lambda i: (0, i, 0)),
            pl.BlockSpec((_BM, D), lambda i: (i, 0)),
            pl.BlockSpec((_BM, D), lambda i: (i, 0)),
            pl.BlockSpec((1, D), lambda i: (0, 0)),
            pl.BlockSpec((D, D), lambda i: (0, 0)),
        ],
        out_specs=pl.BlockSpec((_BM, D), lambda i: (i, 0)),
        out_shape=jax.ShapeDtypeStruct((NP, D), jnp.float32),
    )(aggp, hp, dinvb, b, w)


def _final_body(aggp_ref, hp_ref, dinvb_ref, b_ref, batch_ref, wlin_ref, blin_ref,
                o_ref, pooled_acc, cnt_acc):
    i = pl.program_id(0)

    @pl.when(i == 0)
    def _init():
        pooled_acc[...] = jnp.zeros_like(pooled_acc)
        cnt_acc[...] = jnp.zeros_like(cnt_acc)

    h = (aggp_ref[0] + aggp_ref[1] + hp_ref[...]) * dinvb_ref[...] + b_ref[...]
    h = jnp.maximum(h, 0.0)
    bt = batch_ref[...]
    gids = lax.broadcasted_iota(jnp.int32, (_BM, G), 1)
    m_t = (gids == bt).astype(jnp.float32)
    dn = (((0,), (0,)), ((), ()))
    pooled_acc[...] += lax.dot_general(m_t, h, dn,
                                       preferred_element_type=jnp.float32)
    cnt_acc[...] += lax.dot_general(m_t, jnp.ones((_BM, D), jnp.float32), dn,
                                    preferred_element_type=jnp.float32)

    @pl.when(i == _GRID - 1)
    def _fin():
        pooled = pooled_acc[...] / jnp.maximum(cnt_acc[...], 1.0)
        o_ref[...] = jnp.dot(pooled, wlin_ref[...],
                             preferred_element_type=jnp.float32) + blin_ref[...]


def _final(aggp, hp, dinvb, b, batch2d, wlin, blin):
    return pl.pallas_call(
        _final_body,
        grid=(_GRID,),
        in_specs=[
            pl.BlockSpec((NC, _BM, D), lambda i: (0, i, 0)),
            pl.BlockSpec((_BM, D), lambda i: (i, 0)),
            pl.BlockSpec((_BM, D), lambda i: (i, 0)),
            pl.BlockSpec((1, D), lambda i: (0, 0)),
            pl.BlockSpec((_BM, 1), lambda i: (i, 0)),
            pl.BlockSpec((D, D_OUT), lambda i: (0, 0)),
            pl.BlockSpec((1, D_OUT), lambda i: (0, 0)),
        ],
        out_specs=pl.BlockSpec((G, D_OUT), lambda i: (0, 0)),
        out_shape=jax.ShapeDtypeStruct((G, D_OUT), jnp.float32),
        scratch_shapes=[
            pltpu.VMEM((G, D), jnp.float32),
            pltpu.VMEM((G, D), jnp.float32),
        ],
    )(aggp, hp, dinvb, b, batch2d, wlin, blin)


def kernel(x, edge_index, batch, W1, b1, W2, b2, Wlin, blin):
    pad_e = E_PAD - E
    pad_idx = jnp.full((pad_e,), N, jnp.int32)
    srcw = jnp.concatenate([edge_index[0], pad_idx]).reshape(NW, K, CHUNK)
    dstw = jnp.concatenate([edge_index[1], pad_idx]).reshape(NW, K, CHUNK)
    x_pad = jnp.pad(x, ((0, NP - N), (0, 0)))
    batch2d = jnp.pad(batch, (0, NP - N), constant_values=G).reshape(NP, 1)
    zeros_rows = jnp.zeros((ROWS_PER_TILE, D), jnp.float32)
    zeros_deg = jnp.zeros((ROWS_PER_TILE, DEGW), jnp.float32)
    ones_deg = jnp.ones((CHUNK, DEGW), jnp.float32)
    b1r = b1.reshape(1, D)
    b2r = b2.reshape(1, D)
    blinr = blin.reshape(1, D_OUT)

    degp = _deg_partials(dstw, ones_deg, zeros_deg)
    h1m = _matmul(x_pad, W1)
    dinvb, hp1 = _dinv_and_scale(degp, h1m)
    agg1 = _agg_partials(hp1, srcw, dstw, zeros_rows)
    hp2 = _layer_combine(agg1, hp1, dinvb, b1r, W2)
    agg2 = _agg_partials(hp2, srcw, dstw, zeros_rows)
    return _final(agg2, hp2, dinvb, b2r, batch2d, Wlin, blinr)


# speedup vs baseline: 1.4836x; 1.4836x over previous
import jax
import jax.numpy as jnp
from jax import lax
from jax.experimental import pallas as pl
from jax.experimental.pallas import tpu as pltpu
from jax.experimental.pallas import tpu_sc as plsc

N = 10000
E = 320000
D = 128
D_OUT = 64
G = 64

NP = 10240
NC = 2
NS = 16
NW = NC * NS
CHUNK = 128
K = 79
EPT = K * CHUNK
E_PAD = NW * EPT
ROWS_PER_TILE = NP // NS
DEGW = 128


def _mk_mesh():
    return plsc.VectorSubcoreMesh(core_axis_name="c", subcore_axis_name="s")


def _deg_body(dstw_hbm, ones_hbm, zeros_hbm, out_hbm, dst_v, ones_v, acc_s, sem):
    c = lax.axis_index("c")
    s = lax.axis_index("s")
    wid = s * NC + c
    pltpu.sync_copy(dstw_hbm.at[wid], dst_v)
    pltpu.sync_copy(ones_hbm, ones_v)
    pltpu.sync_copy(zeros_hbm, acc_s.at[pl.ds(s * ROWS_PER_TILE, ROWS_PER_TILE)])
    plsc.subcore_barrier()

    def step(j, carry):
        pltpu.sync_copy(ones_v, acc_s.at[dst_v.at[j]], add=True)
        return carry

    lax.fori_loop(0, K, step, 0)
    plsc.subcore_barrier()
    sl = pl.ds(s * ROWS_PER_TILE, ROWS_PER_TILE)
    pltpu.sync_copy(acc_s.at[sl], out_hbm.at[c, sl])


def _deg_partials(dstw, ones_w, zeros_w):
    return pl.kernel(
        _deg_body,
        out_type=jax.ShapeDtypeStruct((NC, NP, DEGW), jnp.float32),
        mesh=_mk_mesh(),
        scratch_types=[
            pltpu.VMEM((K, CHUNK), jnp.int32),
            pltpu.VMEM((CHUNK, DEGW), jnp.float32),
            pltpu.VMEM_SHARED((NP, DEGW), jnp.float32),
            pltpu.SemaphoreType.DMA,
        ],
    )(dstw, ones_w, zeros_w)


def _agg_body(hp_hbm, srcw_hbm, dstw_hbm, zeros_hbm, out_hbm,
              src_v, dst_v, rows_v, acc_s, sem):
    c = lax.axis_index("c")
    s = lax.axis_index("s")
    wid = s * NC + c
    pltpu.sync_copy(srcw_hbm.at[wid], src_v)
    pltpu.sync_copy(dstw_hbm.at[wid], dst_v)
    pltpu.sync_copy(zeros_hbm, acc_s.at[pl.ds(s * ROWS_PER_TILE, ROWS_PER_TILE)])
    plsc.subcore_barrier()

    def step(j, carry):
        pltpu.async_copy(hp_hbm.at[src_v.at[j]], rows_v, sem).wait()
        pltpu.sync_copy(rows_v, acc_s.at[dst_v.at[j]], add=True)
        return carry

    lax.fori_loop(0, K, step, 0)
    plsc.subcore_barrier()
    sl = pl.ds(s * ROWS_PER_TILE, ROWS_PER_TILE)
    pltpu.sync_copy(acc_s.at[sl], out_hbm.at[c, sl])


def _agg_partials(hp, srcw, dstw, zeros_rows):
    return pl.kernel(
        _agg_body,
        out_type=jax.ShapeDtypeStruct((NC, NP, D), jnp.float32),
        mesh=_mk_mesh(),
        scratch_types=[
            pltpu.VMEM((K, CHUNK), jnp.int32),
            pltpu.VMEM((K, CHUNK), jnp.int32),
            pltpu.VMEM((CHUNK, D), jnp.float32),
            pltpu.VMEM_SHARED((NP, D), jnp.float32),
            pltpu.SemaphoreType.DMA,
        ],
    )(hp, srcw, dstw, zeros_rows)


_BM = 1024
_GRID = NP // _BM


def _mm_body(a_ref, w_ref, o_ref):
    o_ref[...] = jnp.dot(a_ref[...], w_ref[...], preferred_element_type=jnp.float32)


def _matmul(a, w):
    dout = w.shape[1]
    return pl.pallas_call(
        _mm_body,
        grid=(_GRID,),
        in_specs=[
            pl.BlockSpec((_BM, D), lambda i: (i, 0)),
            pl.BlockSpec((D, dout), lambda i: (0, 0)),
        ],
        out_specs=pl.BlockSpec((_BM, dout), lambda i: (i, 0)),
        out_shape=jax.ShapeDtypeStruct((NP, dout), jnp.float32),
    )(a, w)


def _dinv_body(degp_ref, h1m_ref, dinvb_ref, hp1_ref):
    i = pl.program_id(0)
    deg = 1.0 + degp_ref[0, :, 0:1] + degp_ref[1, :, 0:1]
    dinv = lax.rsqrt(deg)
    rid = lax.broadcasted_iota(jnp.int32, (_BM, 1), 0) + i * _BM
    dinv = jnp.where(rid < N, dinv, 0.0)
    dinvb = jnp.broadcast_to(dinv, (_BM, D))
    dinvb_ref[...] = dinvb
    hp1_ref[...] = h1m_ref[...] * dinvb


def _dinv_and_scale(degp, h1m):
    return pl.pallas_call(
        _dinv_body,
        grid=(_GRID,),
        in_specs=[
            pl.BlockSpec((NC, _BM, DEGW), lambda i: (0, i, 0)),
            pl.BlockSpec((_BM, D), lambda i: (i, 0)),
        ],
        out_specs=[
            pl.BlockSpec((_BM, D), lambda i: (i, 0)),
            pl.BlockSpec((_BM, D), lambda i: (i, 0)),
        ],
        out_shape=[
            jax.ShapeDtypeStruct((NP, D), jnp.float32),
            jax.ShapeDtypeStruct((NP, D), jnp.float32),
        ],
    )(degp, h1m)


def _layer_body(aggp_ref, hp_ref, dinvb_ref, b_ref, w_ref, o_ref):
    dinvb = dinvb_ref[...]
    h = (aggp_ref[0] + aggp_ref[1] + hp_ref[...]) * dinvb + b_ref[...]
    h = jnp.maximum(h, 0.0)
    o_ref[...] = jnp.dot(h, w_ref[...], preferred_element_type=jnp.float32) * dinvb


def _layer_combine(aggp, hp, dinvb, b, w):
    return pl.pallas_call(
        _layer_body,
        grid=(_GRID,),
        in_specs=[
            pl.BlockSpec((NC, _BM, D), lambda i: (0, i, 0)),
            pl.BlockSpec((_BM, D), lambda i: (i, 0)),
            pl.BlockSpec((_BM, D), lambda i: (i, 0)),
            pl.BlockSpec((1, D), lambda i: (0, 0)),
            pl.BlockSpec((D, D), lambda i: (0, 0)),
        ],
        out_specs=pl.BlockSpec((_BM, D), lambda i: (i, 0)),
        out_shape=jax.ShapeDtypeStruct((NP, D), jnp.float32),
    )(aggp, hp, dinvb, b, w)


def _final_body(aggp_ref, hp_ref, dinvb_ref, b_ref, batch_ref, wlin_ref, blin_ref,
                o_ref, pooled_acc, cnt_acc):
    i = pl.program_id(0)

    @pl.when(i == 0)
    def _init():
        pooled_acc[...] = jnp.zeros_like(pooled_acc)
        cnt_acc[...] = jnp.zeros_like(cnt_acc)

    h = (aggp_ref[0] + aggp_ref[1] + hp_ref[...]) * dinvb_ref[...] + b_ref[...]
    h = jnp.maximum(h, 0.0)
    bt = batch_ref[...]
    gids = lax.broadcasted_iota(jnp.int32, (_BM, G), 1)
    m_t = (gids == bt).astype(jnp.float32)
    dn = (((0,), (0,)), ((), ()))
    pooled_acc[...] += lax.dot_general(m_t, h, dn,
                                       preferred_element_type=jnp.float32)
    cnt_acc[...] += lax.dot_general(m_t, jnp.ones((_BM, D), jnp.float32), dn,
                                    preferred_element_type=jnp.float32)

    @pl.when(i == _GRID - 1)
    def _fin():
        pooled = pooled_acc[...] / jnp.maximum(cnt_acc[...], 1.0)
        o_ref[...] = jnp.dot(pooled, wlin_ref[...],
                             preferred_element_type=jnp.float32) + blin_ref[...]


def _final(aggp, hp, dinvb, b, batch2d, wlin, blin):
    return pl.pallas_call(
        _final_body,
        grid=(_GRID,),
        in_specs=[
            pl.BlockSpec((NC, _BM, D), lambda i: (0, i, 0)),
            pl.BlockSpec((_BM, D), lambda i: (i, 0)),
            pl.BlockSpec((_BM, D), lambda i: (i, 0)),
            pl.BlockSpec((1, D), lambda i: (0, 0)),
            pl.BlockSpec((_BM, 1), lambda i: (i, 0)),
            pl.BlockSpec((D, D_OUT), lambda i: (0, 0)),
            pl.BlockSpec((1, D_OUT), lambda i: (0, 0)),
        ],
        out_specs=pl.BlockSpec((G, D_OUT), lambda i: (0, 0)),
        out_shape=jax.ShapeDtypeStruct((G, D_OUT), jnp.float32),
        scratch_shapes=[
            pltpu.VMEM((G, D), jnp.float32),
            pltpu.VMEM((G, D), jnp.float32),
        ],
    )(aggp, hp, dinvb, b, batch2d, wlin, blin)


def kernel(x, edge_index, batch, W1, b1, W2, b2, Wlin, blin):
    pad_e = E_PAD - E
    pad_idx = jnp.full((pad_e,), N, jnp.int32)
    srcw = jnp.concatenate([edge_index[0], pad_idx]).reshape(NW, K, CHUNK)
    dstw = jnp.concatenate([edge_index[1], pad_idx]).reshape(NW, K, CHUNK)
    x_pad = jnp.pad(x, ((0, NP - N), (0, 0)))
    batch2d = jnp.pad(batch, (0, NP - N), constant_values=G).reshape(NP, 1)
    zeros_rows = jnp.zeros((ROWS_PER_TILE, D), jnp.float32)
    zeros_deg = jnp.zeros((ROWS_PER_TILE, DEGW), jnp.float32)
    ones_deg = jnp.ones((CHUNK, DEGW), jnp.float32)
    b1r = b1.reshape(1, D)
    b2r = b2.reshape(1, D)
    blinr = blin.reshape(1, D_OUT)

    degp = _deg_partials(dstw, ones_deg, zeros_deg)
    h1m = _matmul(x_pad, W1)
    dinvb, hp1 = _dinv_and_scale(degp, h1m)
    agg1 = _agg_partials(hp1, srcw, dstw, zeros_rows)
    hp2 = _layer_combine(agg1, hp1, dinvb, b1r, W2)
    agg2 = _agg_partials(hp2, srcw, dstw, zeros_rows)
    return _final(agg2, hp2, dinvb, b2r, batch2d, Wlin, blinr)
